# baseline (speedup 1.0000x reference)
import functools

import jax
import jax.numpy as jnp
from jax import lax
from jax.experimental import pallas as pl
from jax.experimental.pallas import tpu as pltpu
from jax.experimental.pallas import tpu_sc as plsc

ATOM_DIM = 256
B, N, E = 256, 128, 256
BB = 32

NC, NS = 2, 16
NW = NC * NS
G_PER_W = B // NW
IDX_CHUNK = 128


def _make_scatter():
    mesh = plsc.VectorSubcoreMesh(core_axis_name="c", subcore_axis_name="s")

    @functools.partial(
        pl.kernel,
        mesh=mesh,
        compiler_params=pltpu.CompilerParams(needs_layout_passes=False),
        out_type=jax.ShapeDtypeStruct((B, N, ATOM_DIM), jnp.float32),
        scratch_types=[
            pltpu.VMEM((E, ATOM_DIM), jnp.float32),
            pltpu.VMEM((N, ATOM_DIM), jnp.float32),
            pltpu.VMEM((E,), jnp.int32),
        ],
    )
    def _scatter(msg_hbm, idx_hbm, out_hbm, msg_v, acc, idx_v):
        cid = lax.axis_index("c")
        sid = lax.axis_index("s")
        wid = sid * NC + cid
        lane = lax.broadcasted_iota(jnp.int32, (16,), 0)

        for gi in range(G_PER_W):
            g = wid * G_PER_W + gi
            pltpu.sync_copy(idx_hbm.at[g], idx_v)
            pltpu.sync_copy(msg_hbm.at[g], msg_v)

            def _zrow(i, _):
                for j in range(ATOM_DIM // 16):
                    acc[i, pl.ds(j * 16, 16)] = jnp.zeros((16,), jnp.float32)
                return 0
            lax.fori_loop(0, N, _zrow, 0)

            def _edge(e, _):
                row_vec = plsc.load_gather(idx_v, [jnp.full((16,), e, jnp.int32)])
                for j in range(ATOM_DIM // 16):
                    plsc.addupdate_scatter(
                        acc, [row_vec, lane + j * 16], msg_v[e, pl.ds(j * 16, 16)]
                    )
                return 0
            lax.fori_loop(0, E, _edge, 0)

            pltpu.sync_copy(acc, out_hbm.at[g])

    return _scatter


_scatter_kernel = _make_scatter()


def _gru_tc_kernel(agg_ref, h_ref, w_ref, u_ref, b_ref, out_ref):
    x = agg_ref[...].reshape(BB * N, ATOM_DIM).astype(jnp.bfloat16)
    h = h_ref[...].reshape(BB * N, ATOM_DIM)
    hb = h.astype(jnp.bfloat16)
    mx = jnp.dot(x, w_ref[...].astype(jnp.bfloat16),
                 preferred_element_type=jnp.float32) + b_ref[0]
    mh = jnp.dot(hb, u_ref[...].astype(jnp.bfloat16),
                 preferred_element_type=jnp.float32) + b_ref[1]
    xz, xr, xh = mx[:, :ATOM_DIM], mx[:, ATOM_DIM:2 * ATOM_DIM], mx[:, 2 * ATOM_DIM:]
    hz, hr, hh_ = mh[:, :ATOM_DIM], mh[:, ATOM_DIM:2 * ATOM_DIM], mh[:, 2 * ATOM_DIM:]
    z = jax.nn.sigmoid(xz + hz)
    r = jax.nn.sigmoid(xr + hr)
    hh = jnp.tanh(xh + r * hh_)
    out_ref[...] = (z * h + (1.0 - z) * hh).reshape(BB, N, ATOM_DIM)


def _gru_tc(agg, atom_state, W, U, b):
    return pl.pallas_call(
        _gru_tc_kernel,
        grid=(B // BB,),
        in_specs=[
            pl.BlockSpec((BB, N, ATOM_DIM), lambda i: (i, 0, 0)),
            pl.BlockSpec((BB, N, ATOM_DIM), lambda i: (i, 0, 0)),
            pl.BlockSpec((ATOM_DIM, 3 * ATOM_DIM), lambda i: (0, 0)),
            pl.BlockSpec((ATOM_DIM, 3 * ATOM_DIM), lambda i: (0, 0)),
            pl.BlockSpec((2, 3 * ATOM_DIM), lambda i: (0, 0)),
        ],
        out_specs=pl.BlockSpec((BB, N, ATOM_DIM), lambda i: (i, 0, 0)),
        out_shape=jax.ShapeDtypeStruct((B, N, ATOM_DIM), jnp.float32),
    )(agg, atom_state, W, U, b)


@jax.jit
def kernel(atom_state, messages, connectivity, W, U, b):
    idx = connectivity[:, :, 1].astype(jnp.int32)
    agg = _scatter_kernel(messages, idx)
    return _gru_tc(agg, atom_state, W, U, b)


# speedup vs baseline: 1.2254x; 1.2254x over previous
import functools

import jax
import jax.numpy as jnp
from jax import lax
from jax.experimental import pallas as pl
from jax.experimental.pallas import tpu as pltpu
from jax.experimental.pallas import tpu_sc as plsc

ATOM_DIM = 256
B, N, E = 256, 128, 256
BB = 32
P_SC = 128

NC, NS = 2, 16
NW = NC * NS
G_PER_W = P_SC // NW
HALF = 128


def _make_scatter():
    mesh = plsc.VectorSubcoreMesh(core_axis_name="c", subcore_axis_name="s")

    @functools.partial(
        pl.kernel,
        mesh=mesh,
        compiler_params=pltpu.CompilerParams(needs_layout_passes=False),
        out_type=jax.ShapeDtypeStruct((P_SC, N * ATOM_DIM), jnp.float32),
        scratch_types=[
            pltpu.VMEM((HALF, ATOM_DIM), jnp.float32),
            pltpu.VMEM((HALF, ATOM_DIM), jnp.float32),
            pltpu.VMEM((HALF, 16), jnp.int32),
            pltpu.VMEM((HALF, 16), jnp.int32),
            pltpu.VMEM((N * ATOM_DIM,), jnp.float32),
            pltpu.SemaphoreType.DMA,
            pltpu.SemaphoreType.DMA,
            pltpu.SemaphoreType.DMA,
            pltpu.SemaphoreType.DMA,
        ],
    )
    def _scatter(msg_hbm, addr_hbm, out_hbm,
                 m0, m1, a0, a1, acc, sm0, sm1, sa0, sa1):
        cid = lax.axis_index("c")
        sid = lax.axis_index("s")
        wid = sid * NC + cid
        g_base = wid * G_PER_W
        mbuf, abuf = [m0, m1], [a0, a1]
        msem, asem = [sm0, sm1], [sa0, sa1]
        n_chunks = 2 * G_PER_W

        def chunk_refs(t):
            g = g_base + t // 2
            sl = pl.ds((t % 2) * HALF, HALF)
            return msg_hbm.at[g, sl], addr_hbm.at[g, sl]

        m_src, a_src = chunk_refs(0)
        pltpu.async_copy(m_src, m0, sm0)
        pltpu.async_copy(a_src, a0, sa0)

        for t in range(n_chunks):
            b = t % 2
            g, h = g_base + t // 2, t % 2
            if t + 1 < n_chunks:
                nb = (t + 1) % 2
                m_src, a_src = chunk_refs(t + 1)
                pltpu.async_copy(m_src, mbuf[nb], msem[nb])
                pltpu.async_copy(a_src, abuf[nb], asem[nb])

            if h == 0:
                def _zero(i, _):
                    acc[pl.ds(i * 16, 16)] = jnp.zeros((16,), jnp.float32)
                    return 0
                lax.fori_loop(0, N * ATOM_DIM // 16, _zero, 0)

            m_src, a_src = chunk_refs(t)
            pltpu.make_async_copy(m_src, mbuf[b], msem[b]).wait()
            pltpu.make_async_copy(a_src, abuf[b], asem[b]).wait()

            mb, ab = mbuf[b], abuf[b]

            def _edge(e, _):
                adv = ab[e]
                for j in range(ATOM_DIM // 16):
                    plsc.addupdate_scatter(
                        acc, [adv + j * 16], mb[e, pl.ds(j * 16, 16)]
                    )
                return 0
            lax.fori_loop(0, HALF, _edge, 0)

            if h == 1:
                pltpu.sync_copy(acc, out_hbm.at[g])

    return _scatter


_scatter_kernel = _make_scatter()


def _onehot_gru_kernel(idx_ref, msg_ref, h_ref, w_ref, u_ref, b_ref, out_ref):
    idx = idx_ref[:, 0, :]
    node_ids = jax.lax.broadcasted_iota(jnp.int32, (BB, N, E), 1)
    onehot = (idx[:, None, :] == node_ids).astype(jnp.float32)
    agg = jax.lax.dot_general(
        onehot, msg_ref[...],
        dimension_numbers=(((2,), (1,)), ((0,), (0,))),
        preferred_element_type=jnp.float32,
    )
    _gru_math(agg, h_ref, w_ref, u_ref, b_ref, out_ref)


def _agg_gru_kernel(agg_ref, h_ref, w_ref, u_ref, b_ref, dummy_ref, out_ref):
    del dummy_ref
    _gru_math(agg_ref[...], h_ref, w_ref, u_ref, b_ref, out_ref)


def _gru_math(agg, h_ref, w_ref, u_ref, b_ref, out_ref):
    x = agg.reshape(BB * N, ATOM_DIM).astype(jnp.bfloat16)
    h = h_ref[...].reshape(BB * N, ATOM_DIM)
    hb = h.astype(jnp.bfloat16)
    mx = jnp.dot(x, w_ref[...].astype(jnp.bfloat16),
                 preferred_element_type=jnp.float32) + b_ref[0]
    mh = jnp.dot(hb, u_ref[...].astype(jnp.bfloat16),
                 preferred_element_type=jnp.float32) + b_ref[1]
    xz, xr, xh = mx[:, :ATOM_DIM], mx[:, ATOM_DIM:2 * ATOM_DIM], mx[:, 2 * ATOM_DIM:]
    hz, hr, hh_ = mh[:, :ATOM_DIM], mh[:, ATOM_DIM:2 * ATOM_DIM], mh[:, 2 * ATOM_DIM:]
    z = jax.nn.sigmoid(xz + hz)
    r = jax.nn.sigmoid(xr + hr)
    hh = jnp.tanh(xh + r * hh_)
    out_ref[...] = (z * h + (1.0 - z) * hh).reshape(BB, N, ATOM_DIM)


_W_SPEC = pl.BlockSpec((ATOM_DIM, 3 * ATOM_DIM), lambda i: (0, 0))
_B_SPEC = pl.BlockSpec((2, 3 * ATOM_DIM), lambda i: (0, 0))
_OFF = P_SC // BB


def _stage1(tgt_idx, messages, atom_state, W, U, b):
    return pl.pallas_call(
        _onehot_gru_kernel,
        grid=((B - P_SC) // BB,),
        in_specs=[
            pl.BlockSpec((BB, 1, E), lambda i: (i + _OFF, 0, 0)),
            pl.BlockSpec((BB, E, ATOM_DIM), lambda i: (i + _OFF, 0, 0)),
            pl.BlockSpec((BB, N, ATOM_DIM), lambda i: (i + _OFF, 0, 0)),
            _W_SPEC, _W_SPEC, _B_SPEC,
        ],
        out_specs=pl.BlockSpec((BB, N, ATOM_DIM), lambda i: (i + _OFF, 0, 0)),
        out_shape=jax.ShapeDtypeStruct((B, N, ATOM_DIM), jnp.float32),
    )(tgt_idx, messages, atom_state, W, U, b)


def _stage2(agg, atom_state, W, U, b, out_partial):
    return pl.pallas_call(
        _agg_gru_kernel,
        grid=(P_SC // BB,),
        in_specs=[
            pl.BlockSpec((BB, N, ATOM_DIM), lambda i: (i, 0, 0)),
            pl.BlockSpec((BB, N, ATOM_DIM), lambda i: (i, 0, 0)),
            _W_SPEC, _W_SPEC, _B_SPEC,
            pl.BlockSpec((BB, N, ATOM_DIM), lambda i: (0, 0, 0)),
        ],
        out_specs=pl.BlockSpec((BB, N, ATOM_DIM), lambda i: (i, 0, 0)),
        out_shape=jax.ShapeDtypeStruct((B, N, ATOM_DIM), jnp.float32),
        input_output_aliases={5: 0},
    )(agg, atom_state, W, U, b, out_partial)


@jax.jit
def kernel(atom_state, messages, connectivity, W, U, b):
    idx = connectivity[:, :, 1].astype(jnp.int32)
    lane = jnp.arange(16, dtype=jnp.int32)
    addr = idx[:P_SC, :, None] * ATOM_DIM + lane[None, None, :]
    agg = _scatter_kernel(messages, addr).reshape(P_SC, N, ATOM_DIM)
    out_partial = _stage1(idx.reshape(B, 1, E), messages, atom_state, W, U, b)
    return _stage2(agg, atom_state, W, U, b, out_partial)


# speedup vs baseline: 2.8921x; 2.3600x over previous
import functools

import jax
import jax.numpy as jnp
from jax import lax
from jax.experimental import pallas as pl
from jax.experimental.pallas import tpu as pltpu
from jax.experimental.pallas import tpu_sc as plsc

ATOM_DIM = 256
B, N, E = 256, 128, 256
BB = 32
P_SC = 32

NC, NS = 2, 16
NW = NC * NS
G_PER_W = P_SC // NW
HALF = 128


def _make_scatter():
    mesh = plsc.VectorSubcoreMesh(core_axis_name="c", subcore_axis_name="s")

    @functools.partial(
        pl.kernel,
        mesh=mesh,
        compiler_params=pltpu.CompilerParams(needs_layout_passes=False),
        out_type=jax.ShapeDtypeStruct((P_SC, N, ATOM_DIM), jnp.float32),
        scratch_types=[
            pltpu.VMEM((HALF, ATOM_DIM), jnp.float32),
            pltpu.VMEM((HALF, ATOM_DIM), jnp.float32),
            pltpu.VMEM((E,), jnp.int32),
            pltpu.VMEM((N, ATOM_DIM), jnp.float32),
            pltpu.SemaphoreType.DMA,
            pltpu.SemaphoreType.DMA,
            pltpu.SemaphoreType.DMA,
        ],
    )
    def _scatter(msg_hbm, idx_hbm, out_hbm, m0, m1, idx_v, acc, sm0, sm1, si):
        cid = lax.axis_index("c")
        sid = lax.axis_index("s")
        wid = sid * NC + cid
        g_base = wid * G_PER_W
        mbuf, msem = [m0, m1], [sm0, sm1]
        lane = lax.broadcasted_iota(jnp.int32, (16,), 0)
        n_chunks = 2 * G_PER_W

        def msg_src(t):
            return msg_hbm.at[g_base + t // 2, pl.ds((t % 2) * HALF, HALF)]

        pltpu.async_copy(msg_src(0), m0, sm0)
        pltpu.async_copy(idx_hbm.at[g_base], idx_v, si)

        for t in range(n_chunks):
            b = t % 2
            g, h = g_base + t // 2, t % 2
            if t + 1 < n_chunks:
                pltpu.async_copy(msg_src(t + 1), mbuf[(t + 1) % 2], msem[(t + 1) % 2])

            if h == 0:
                def _zrow(i, _):
                    for j in range(ATOM_DIM // 16):
                        acc[i, pl.ds(j * 16, 16)] = jnp.zeros((16,), jnp.float32)
                    return 0
                lax.fori_loop(0, N, _zrow, 0)
                pltpu.make_async_copy(idx_hbm.at[g], idx_v, si).wait()

            pltpu.make_async_copy(msg_src(t), mbuf[b], msem[b]).wait()
            mb = mbuf[b]

            def _edge(el, _):
                e = el + h * HALF
                row_vec = plsc.load_gather(idx_v, [jnp.full((16,), e, jnp.int32)])
                for j in range(ATOM_DIM // 16):
                    plsc.addupdate_scatter(
                        acc, [row_vec, lane + j * 16], mb[el, pl.ds(j * 16, 16)]
                    )
                return 0
            lax.fori_loop(0, HALF, _edge, 0)

            if h == 1:
                pltpu.sync_copy(acc, out_hbm.at[g])
                if t + 1 < n_chunks:
                    pltpu.async_copy(idx_hbm.at[g + 1], idx_v, si)

    return _scatter


_scatter_kernel = _make_scatter()


def _onehot_gru_kernel(idx_ref, msg_ref, h_ref, w_ref, u_ref, b_ref, out_ref):
    idx = idx_ref[:, 0, :]
    node_ids = jax.lax.broadcasted_iota(jnp.int32, (BB, N, E), 1)
    onehot = (idx[:, None, :] == node_ids).astype(jnp.float32)
    agg = jax.lax.dot_general(
        onehot, msg_ref[...],
        dimension_numbers=(((2,), (1,)), ((0,), (0,))),
        preferred_element_type=jnp.float32,
    )
    _gru_math(agg, h_ref, w_ref, u_ref, b_ref, out_ref)


def _agg_gru_kernel(agg_ref, h_ref, w_ref, u_ref, b_ref, dummy_ref, out_ref):
    del dummy_ref
    _gru_math(agg_ref[...], h_ref, w_ref, u_ref, b_ref, out_ref)


def _gru_math(agg, h_ref, w_ref, u_ref, b_ref, out_ref):
    x = agg.reshape(BB * N, ATOM_DIM).astype(jnp.bfloat16)
    h = h_ref[...].reshape(BB * N, ATOM_DIM)
    hb = h.astype(jnp.bfloat16)
    mx = jnp.dot(x, w_ref[...].astype(jnp.bfloat16),
                 preferred_element_type=jnp.float32) + b_ref[0]
    mh = jnp.dot(hb, u_ref[...].astype(jnp.bfloat16),
                 preferred_element_type=jnp.float32) + b_ref[1]
    xz, xr, xh = mx[:, :ATOM_DIM], mx[:, ATOM_DIM:2 * ATOM_DIM], mx[:, 2 * ATOM_DIM:]
    hz, hr, hh_ = mh[:, :ATOM_DIM], mh[:, ATOM_DIM:2 * ATOM_DIM], mh[:, 2 * ATOM_DIM:]
    z = jax.nn.sigmoid(xz + hz)
    r = jax.nn.sigmoid(xr + hr)
    hh = jnp.tanh(xh + r * hh_)
    out_ref[...] = (z * h + (1.0 - z) * hh).reshape(BB, N, ATOM_DIM)


_W_SPEC = pl.BlockSpec((ATOM_DIM, 3 * ATOM_DIM), lambda i: (0, 0))
_B_SPEC = pl.BlockSpec((2, 3 * ATOM_DIM), lambda i: (0, 0))
_OFF = P_SC // BB


def _stage1(tgt_idx, messages, atom_state, W, U, b):
    return pl.pallas_call(
        _onehot_gru_kernel,
        grid=((B - P_SC) // BB,),
        in_specs=[
            pl.BlockSpec((BB, 1, E), lambda i: (i + _OFF, 0, 0)),
            pl.BlockSpec((BB, E, ATOM_DIM), lambda i: (i + _OFF, 0, 0)),
            pl.BlockSpec((BB, N, ATOM_DIM), lambda i: (i + _OFF, 0, 0)),
            _W_SPEC, _W_SPEC, _B_SPEC,
        ],
        out_specs=pl.BlockSpec((BB, N, ATOM_DIM), lambda i: (i + _OFF, 0, 0)),
        out_shape=jax.ShapeDtypeStruct((B, N, ATOM_DIM), jnp.float32),
    )(tgt_idx, messages, atom_state, W, U, b)


def _stage2(agg, atom_state, W, U, b, out_partial):
    return pl.pallas_call(
        _agg_gru_kernel,
        grid=(P_SC // BB,),
        in_specs=[
            pl.BlockSpec((BB, N, ATOM_DIM), lambda i: (i, 0, 0)),
            pl.BlockSpec((BB, N, ATOM_DIM), lambda i: (i, 0, 0)),
            _W_SPEC, _W_SPEC, _B_SPEC,
            pl.BlockSpec((BB, N, ATOM_DIM), lambda i: (0, 0, 0)),
        ],
        out_specs=pl.BlockSpec((BB, N, ATOM_DIM), lambda i: (i, 0, 0)),
        out_shape=jax.ShapeDtypeStruct((B, N, ATOM_DIM), jnp.float32),
        input_output_aliases={5: 0},
    )(agg, atom_state, W, U, b, out_partial)


@jax.jit
def kernel(atom_state, messages, connectivity, W, U, b):
    idx = connectivity[:, :, 1].astype(jnp.int32)
    agg = _scatter_kernel(messages, idx)
    out_partial = _stage1(idx.reshape(B, 1, E), messages, atom_state, W, U, b)
    return _stage2(agg, atom_state, W, U, b, out_partial)
